# Initial kernel scaffold; baseline (speedup 1.0000x reference)
import jax
import jax.numpy as jnp
from jax.experimental import pallas as pl


def kernel(x, edge_attr, edge_paths, edge_vector):
    raise NotImplementedError("write your pallas kernel here")


# speedup vs baseline: 90.5196x; 90.5196x over previous
import functools

import numpy as np
import jax
import jax.numpy as jnp
from jax import lax
from jax.experimental import pallas as pl
from jax.experimental.pallas import tpu as pltpu
from jax.experimental.pallas import tpu_sc as plsc

N = 2048
E = 65536
D = 128
L = 5

NUM_CORES = 2
NUM_SUBCORES = 16
NW = NUM_CORES * NUM_SUBCORES
NP = N * N
PAIRS_PER_W = NP // NW
CHUNK = 2048
CHUNKS = PAIRS_PER_W // CHUNK
VECS = CHUNK // 16

_HI_MASK = np.int32(-65536)


def _scores_body(ev_ref, ea_ref, out_ref):
    out_ref[...] = lax.dot_general(
        ev_ref[...], ea_ref[...],
        (((1,), (1,)), ((), ())),
        preferred_element_type=jnp.float32,
    )


def _tc_scores(ev_pad, edge_attr):
    blk = 512
    grid = E // blk
    return pl.pallas_call(
        _scores_body,
        grid=(grid,),
        in_specs=[
            pl.BlockSpec((8, D), lambda j: (0, 0)),
            pl.BlockSpec((blk, D), lambda j: (j, 0)),
        ],
        out_specs=pl.BlockSpec((8, blk), lambda j: (0, j)),
        out_shape=jax.ShapeDtypeStruct((8, E), jnp.float32),
    )(ev_pad, edge_attr)


def _sc_body(ep_ref, t01_ref, t23_ref, t4_ref, out_ref,
             tab_ref, tab4_ref, idx_ref, part_ref, prev_ref):
    wid = lax.axis_index("s") * NUM_CORES + lax.axis_index("c")
    start = wid * PAIRS_PER_W
    iota5 = lax.iota(jnp.int32, 16) * 5

    pltpu.sync_copy(t01_ref, tab_ref)

    def chunk_a(c, carry):
        base = start + c * CHUNK
        pltpu.sync_copy(ep_ref.at[pl.ds(base * 5, 5 * CHUNK)], idx_ref)

        def inner(i, carry2):
            addr = iota5 + i * 80
            e0 = plsc.load_gather(idx_ref, [addr])
            e1 = plsc.load_gather(idx_ref, [addr + 1])
            w0 = plsc.load_gather(tab_ref, [e0])
            w1 = plsc.load_gather(tab_ref, [e1])
            s0 = plsc.bitcast(lax.shift_left(w0, 16), jnp.float32)
            s1 = plsc.bitcast(lax.bitwise_and(w1, _HI_MASK), jnp.float32)
            part_ref[pl.ds(i * 16, 16)] = s0 + s1
            return carry2

        lax.fori_loop(0, VECS, inner, 0)
        pltpu.sync_copy(part_ref, out_ref.at[pl.ds(base, CHUNK)])
        return carry

    lax.fori_loop(0, CHUNKS, chunk_a, 0)

    pltpu.sync_copy(t23_ref, tab_ref)
    pltpu.sync_copy(t4_ref, tab4_ref)

    def chunk_b(c, carry):
        base = start + c * CHUNK
        pltpu.sync_copy(ep_ref.at[pl.ds(base * 5, 5 * CHUNK)], idx_ref)
        pltpu.sync_copy(out_ref.at[pl.ds(base, CHUNK)], prev_ref)

        def inner(i, carry2):
            addr = iota5 + i * 80
            e2 = plsc.load_gather(idx_ref, [addr + 2])
            e3 = plsc.load_gather(idx_ref, [addr + 3])
            e4 = plsc.load_gather(idx_ref, [addr + 4])
            w2 = plsc.load_gather(tab_ref, [e2])
            w3 = plsc.load_gather(tab_ref, [e3])
            w4 = plsc.load_gather(tab4_ref, [lax.shift_right_logical(e4, 1)])
            s2 = plsc.bitcast(lax.shift_left(w2, 16), jnp.float32)
            s3 = plsc.bitcast(lax.bitwise_and(w3, _HI_MASK), jnp.float32)
            shift = lax.shift_left(1 - lax.bitwise_and(e4, 1), 4)
            s4 = plsc.bitcast(
                lax.bitwise_and(lax.shift_left(w4, shift), _HI_MASK),
                jnp.float32)
            prev = prev_ref[pl.ds(i * 16, 16)]
            part_ref[pl.ds(i * 16, 16)] = prev + s2 + s3 + s4
            return carry2

        lax.fori_loop(0, VECS, inner, 0)
        pltpu.sync_copy(part_ref, out_ref.at[pl.ds(base, CHUNK)])
        return carry

    lax.fori_loop(0, CHUNKS, chunk_b, 0)


def _sc_gather(ep_flat, t01, t23, t4):
    mesh = plsc.VectorSubcoreMesh(core_axis_name="c", subcore_axis_name="s")
    kern = functools.partial(
        pl.kernel,
        mesh=mesh,
        compiler_params=pltpu.CompilerParams(needs_layout_passes=False),
        out_type=jax.ShapeDtypeStruct((NP,), jnp.float32),
        scratch_types=[
            pltpu.VMEM((E,), jnp.int32),
            pltpu.VMEM((E // 2,), jnp.int32),
            pltpu.VMEM((5 * CHUNK,), jnp.int32),
            pltpu.VMEM((CHUNK,), jnp.float32),
            pltpu.VMEM((CHUNK,), jnp.float32),
        ],
    )(_sc_body)
    return kern(ep_flat, t01, t23, t4)


def kernel(x, edge_attr, edge_paths, edge_vector):
    assert edge_attr.shape == (E, D) and edge_paths.shape == (N, N, L)
    ev_pad = jnp.zeros((8, D), jnp.float32).at[:L].set(edge_vector / L)
    scores_t = _tc_scores(ev_pad, edge_attr)

    b = scores_t.astype(jnp.bfloat16)
    u = lax.bitcast_convert_type(b, jnp.uint16).astype(jnp.uint32)
    t01 = lax.bitcast_convert_type(u[0] | (u[1] << 16), jnp.int32)
    t23 = lax.bitcast_convert_type(u[2] | (u[3] << 16), jnp.int32)
    t4 = lax.bitcast_convert_type(u[4][0::2] | (u[4][1::2] << 16), jnp.int32)

    ep_flat = edge_paths.reshape(-1)
    cij_flat = _sc_gather(ep_flat, t01, t23, t4)
    return cij_flat.reshape(N, N)


# speedup vs baseline: 776.1508x; 8.5744x over previous
import functools

import numpy as np
import jax
import jax.numpy as jnp
from jax import lax
from jax.experimental import pallas as pl
from jax.experimental.pallas import tpu as pltpu
from jax.experimental.pallas import tpu_sc as plsc

N = 2048
E = 65536
D = 128
L = 5

NUM_CORES = 2
NUM_SUBCORES = 16
NW = NUM_CORES * NUM_SUBCORES
TR = N // 8
TR_PER_W = TR // NW
LC = 256
LCHUNKS = N // LC
CHUNKS = TR_PER_W * LCHUNKS
VECS = 8 * LC // 16

_HI_MASK = np.int32(-65536)


def _scores_body(ev_ref, ea_ref, out_ref):
    out_ref[...] = lax.dot_general(
        ev_ref[...], ea_ref[...],
        (((1,), (1,)), ((), ())),
        preferred_element_type=jnp.float32,
    )


def _tc_scores(ev_pad, edge_attr):
    blk = 512
    return pl.pallas_call(
        _scores_body,
        grid=(E // blk,),
        in_specs=[
            pl.BlockSpec((8, D), lambda j: (0, 0)),
            pl.BlockSpec((blk, D), lambda j: (j, 0)),
        ],
        out_specs=pl.BlockSpec((8, blk), lambda j: (0, j)),
        out_shape=jax.ShapeDtypeStruct((8, E), jnp.float32),
    )(ev_pad, edge_attr)


def _unpack_lo(w):
    return plsc.bitcast(lax.shift_left(w, 16), jnp.float32)


def _unpack_hi(w):
    return plsc.bitcast(lax.bitwise_and(w, _HI_MASK), jnp.float32)


def _sc_body(ep_ref, t01_ref, t23_ref, t4_ref, out_ref,
             tab_ref, tab4_ref, inb_ref, prev_ref, outb_ref,
             sem_in0, sem_in1, sem_out0, sem_out1):
    wid = lax.axis_index("s") * NUM_CORES + lax.axis_index("c")
    row0 = wid * TR_PER_W * 8
    sems_in = (sem_in0, sem_in1)
    sems_out = (sem_out0, sem_out1)

    def slab(c):
        r8 = row0 + (c // LCHUNKS) * 8
        lo = (c % LCHUNKS) * LC
        return r8, lo

    def run_phase(planes, with_prev):
        nplanes = len(planes)

        def start_in(c, b):
            r8, lo = slab(c)
            for k, p in enumerate(planes):
                pltpu.async_copy(
                    ep_ref.at[p, pl.ds(r8, 8), pl.ds(lo, LC)],
                    inb_ref.at[b, k], sems_in[b])
            if with_prev:
                pltpu.async_copy(
                    out_ref.at[pl.ds(r8, 8), pl.ds(lo, LC)],
                    prev_ref.at[b], sems_in[b])

        def wait_in(c, b):
            r8, lo = slab(c)
            for k in range(nplanes):
                pltpu.make_async_copy(
                    ep_ref.at[planes[0], pl.ds(r8, 8), pl.ds(lo, LC)],
                    inb_ref.at[b, k], sems_in[b]).wait()
            if with_prev:
                pltpu.make_async_copy(
                    out_ref.at[pl.ds(r8, 8), pl.ds(lo, LC)],
                    prev_ref.at[b], sems_in[b]).wait()

        def start_out(c, b):
            r8, lo = slab(c)
            pltpu.async_copy(
                outb_ref.at[b],
                out_ref.at[pl.ds(r8, 8), pl.ds(lo, LC)], sems_out[b])

        def wait_out(c, b):
            r8, lo = slab(c)
            pltpu.make_async_copy(
                outb_ref.at[b],
                out_ref.at[pl.ds(r8, 8), pl.ds(lo, LC)], sems_out[b]).wait()

        def compute(b):
            def inner(v, carry):
                ri = lax.shift_right_logical(v, 4)
                ci = lax.shift_left(lax.bitwise_and(v, 15), 4)
                if with_prev:
                    i2 = inb_ref[b, 0, ri, pl.ds(ci, 16)]
                    i3 = inb_ref[b, 1, ri, pl.ds(ci, 16)]
                    i4 = inb_ref[b, 2, ri, pl.ds(ci, 16)]
                    w2 = plsc.load_gather(tab_ref, [i2])
                    w3 = plsc.load_gather(tab_ref, [i3])
                    w4 = plsc.load_gather(
                        tab4_ref, [lax.shift_right_logical(i4, 1)])
                    sh = lax.shift_left(1 - lax.bitwise_and(i4, 1), 4)
                    s4 = plsc.bitcast(
                        lax.bitwise_and(lax.shift_left(w4, sh), _HI_MASK),
                        jnp.float32)
                    acc = (prev_ref[b, ri, pl.ds(ci, 16)]
                           + _unpack_lo(w2) + _unpack_hi(w3) + s4)
                else:
                    i0 = inb_ref[b, 0, ri, pl.ds(ci, 16)]
                    i1 = inb_ref[b, 1, ri, pl.ds(ci, 16)]
                    w0 = plsc.load_gather(tab_ref, [i0])
                    w1 = plsc.load_gather(tab_ref, [i1])
                    acc = _unpack_lo(w0) + _unpack_hi(w1)
                outb_ref[b, ri, pl.ds(ci, 16)] = acc
                return carry

            lax.fori_loop(0, VECS, inner, 0)

        def step(c, b, first):
            wait_in(c, b)
            if not first:
                wait_out(c - 2, b)
            compute(b)
            start_out(c, b)

        start_in(0, 0)
        start_in(1, 1)
        step(0, 0, True)
        start_in(2, 0)
        step(1, 1, True)
        start_in(3, 1)

        def main(c2, carry):
            c = 2 * c2
            step(c, 0, False)
            start_in(c + 2, 0)
            step(c + 1, 1, False)
            start_in(c + 3, 1)
            return carry

        lax.fori_loop(1, CHUNKS // 2 - 1, main, 0)

        step(CHUNKS - 2, 0, False)
        step(CHUNKS - 1, 1, False)
        wait_out(CHUNKS - 2, 0)
        wait_out(CHUNKS - 1, 1)

    pltpu.sync_copy(t01_ref, tab_ref)
    run_phase((0, 1), False)

    pltpu.sync_copy(t23_ref, tab_ref)
    pltpu.sync_copy(t4_ref, tab4_ref)
    run_phase((2, 3, 4), True)


def _sc_gather(ep_t, t01, t23, t4):
    mesh = plsc.VectorSubcoreMesh(core_axis_name="c", subcore_axis_name="s")
    kern = functools.partial(
        pl.kernel,
        mesh=mesh,
        compiler_params=pltpu.CompilerParams(needs_layout_passes=False),
        out_type=jax.ShapeDtypeStruct((N, N), jnp.float32),
        scratch_types=[
            pltpu.VMEM((E,), jnp.int32),
            pltpu.VMEM((E // 2,), jnp.int32),
            pltpu.VMEM((2, 3, 8, LC), jnp.int32),
            pltpu.VMEM((2, 8, LC), jnp.float32),
            pltpu.VMEM((2, 8, LC), jnp.float32),
            pltpu.SemaphoreType.DMA,
            pltpu.SemaphoreType.DMA,
            pltpu.SemaphoreType.DMA,
            pltpu.SemaphoreType.DMA,
        ],
    )(_sc_body)
    return kern(ep_t, t01, t23, t4)


def kernel(x, edge_attr, edge_paths, edge_vector):
    assert edge_attr.shape == (E, D) and edge_paths.shape == (N, N, L)
    ev_pad = jnp.zeros((8, D), jnp.float32).at[:L].set(edge_vector / L)
    scores_t = _tc_scores(ev_pad, edge_attr)

    b = scores_t.astype(jnp.bfloat16)
    u = lax.bitcast_convert_type(b, jnp.uint16).astype(jnp.uint32)
    t01 = lax.bitcast_convert_type(u[0] | (u[1] << 16), jnp.int32)
    t23 = lax.bitcast_convert_type(u[2] | (u[3] << 16), jnp.int32)
    t4 = lax.bitcast_convert_type(u[4][0::2] | (u[4][1::2] << 16), jnp.int32)

    ep_t = jnp.transpose(edge_paths, (2, 0, 1))
    return _sc_gather(ep_t, t01, t23, t4)
